# baseline (speedup 1.0000x reference)
import functools

import numpy as np
import jax
import jax.numpy as jnp
from jax import lax
from jax.experimental import pallas as pl
from jax.experimental.pallas import tpu as pltpu
from jax.experimental.pallas import tpu_sc as plsc

_F32 = jnp.float32

_NC, _NS, _LANES = 2, 16, 16
_NW = _NC * _NS
_CH = 32
_P = 64


def _relu(x):
    return jnp.maximum(x, 0.0)


def _dot(a, b):
    return jnp.dot(a, b, preferred_element_type=_F32)


def _tcab_body(x_ref, wt_ref, wv_ref, bv_ref, table_ref, v_ref):
    x = x_ref[0]
    table_ref[0] = _dot(x, wt_ref[...])
    v_ref[0] = _dot(x, wv_ref[...]) + bv_ref[...]


def _tc_ab(x, wt, wv, bv):
    B, R, CIN4 = x.shape
    return pl.pallas_call(
        _tcab_body,
        grid=(B,),
        in_specs=[
            pl.BlockSpec((1, R, CIN4), lambda b: (b, 0, 0)),
            pl.BlockSpec((CIN4, 128), lambda b: (0, 0)),
            pl.BlockSpec((CIN4, 128), lambda b: (0, 0)),
            pl.BlockSpec((1, 128), lambda b: (0, 0)),
        ],
        out_specs=[
            pl.BlockSpec((1, R, 128), lambda b: (b, 0, 0)),
            pl.BlockSpec((1, R, 128), lambda b: (b, 0, 0)),
        ],
        out_shape=[
            jax.ShapeDtypeStruct((B, R, 128), _F32),
            jax.ShapeDtypeStruct((B, R, 128), _F32),
        ],
    )(x, wt, wv, bv)


def _blockdiag4(w):
    cin = w.shape[0]
    out = jnp.zeros((4 * cin, 128), _F32)
    for q in range(4):
        out = out.at[q * cin:(q + 1) * cin, q * _CH:(q + 1) * _CH].set(w)
    return out


def _sc_edge_gather(table_flat, idx_rows, v_flat, B, N, K, emit_extras):
    pts_w = B * N // _NW
    n_chunks = pts_w // _P
    rpc = _P * K
    ng = rpc // 128
    per_b_w = _NW // B
    mesh = plsc.VectorSubcoreMesh(core_axis_name="c", subcore_axis_name="s")

    out_type = []
    if emit_extras:
        out_type.append(jax.ShapeDtypeStruct((B * N // 4, 128), _F32))
    out_type.append(jax.ShapeDtypeStruct((B, 24, N), _F32))
    if emit_extras:
        out_type.append(jax.ShapeDtypeStruct((B, 3, N * K), _F32))

    idx_rows_w = pts_w * K // 128
    scratch_types = [
        pltpu.VMEM((idx_rows_w, 128), jnp.int32),
        pltpu.VMEM((rpc, _CH), _F32),
        pltpu.VMEM((rpc, _CH), _F32),
        pltpu.VMEM((_P // 4, 128), _F32),
        pltpu.VMEM((_P // 4, 128), _F32),
        pltpu.VMEM((_P // 4, 128), _F32),
        pltpu.VMEM((_P // 4, 128), _F32),
        pltpu.VMEM((24, pts_w), _F32),
        pltpu.SemaphoreType.DMA,
        pltpu.SemaphoreType.DMA,
        pltpu.SemaphoreType.DMA,
        pltpu.SemaphoreType.DMA,
    ]
    if emit_extras:
        scratch_types.append(pltpu.VMEM((3, rpc), _F32))
        scratch_types.append(pltpu.VMEM((3, rpc), _F32))

    @functools.partial(
        pl.kernel, out_type=tuple(out_type), mesh=mesh,
        scratch_types=scratch_types,
        compiler_params=pltpu.CompilerParams(use_tc_tiling_on_sc=False,
                                             needs_layout_passes=False))
    def k(table, idxg, vrows, *rest):
        if emit_extras:
            rows_out, cm_out, knn_out = rest[0], rest[1], rest[2]
            (idxall, rowbuf_a, rowbuf_b, vbuf_a, vbuf_b, abuf_a, abuf_b,
             cmbuf, sem, sem_v, sem_wa, sem_wb, kbuf_a, kbuf_b) = rest[3:]
            kbufs = (kbuf_a, kbuf_b)
        else:
            cm_out = rest[0]
            (idxall, rowbuf_a, rowbuf_b, vbuf_a, vbuf_b, abuf_a, abuf_b,
             cmbuf, sem, sem_v, sem_wa, sem_wb) = rest[1:]

        rowbufs = (rowbuf_a, rowbuf_b)
        vbufs = (vbuf_a, vbuf_b)
        abufs = (abuf_a, abuf_b)
        sem_ws = (sem_wa, sem_wb)
        wid = lax.axis_index("s") * _NC + lax.axis_index("c")
        base_pt = wid * pts_w
        b = wid // per_b_w
        pt_in_b = base_pt - b * N
        iot = lax.iota(jnp.int32, _LANES)

        irow0 = pl.multiple_of(base_pt * K // 128, 8)
        pltpu.sync_copy(idxg.at[pl.ds(irow0, idx_rows_w)], idxall)
        bn = b * N

        def badd_body(r, _):
            for j8 in range(128 // _LANES):
                sl = pl.ds(j8 * _LANES, _LANES)
                idxall[r, sl] = idxall[r, sl] + bn
            return 0

        lax.fori_loop(0, idx_rows_w, badd_body, 0)

        def issue(ci, par):
            pt0 = pl.multiple_of(base_pt + ci * _P, _P)
            for j in range(ng):
                pltpu.async_copy(table.at[idxall.at[ci * ng + j]],
                                 rowbufs[par].at[pl.ds(j * 128, 128)], sem)
            pltpu.async_copy(vrows.at[pl.ds(pt0 // 4, _P // 4)], vbufs[par],
                             sem_v)

        def compute(ci, par):
            rowbuf, vbuf, abuf = rowbufs[par], vbufs[par], abufs[par]
            pt0 = pl.multiple_of(base_pt + ci * _P, _P)
            pltpu.make_async_copy(table.at[pl.ds(0, rpc)], rowbuf, sem).wait()
            pltpu.make_async_copy(vrows.at[pl.ds(0, _P // 4)], vbuf,
                                  sem_v).wait()
            if emit_extras:
                @pl.when(ci >= 2)
                def _():
                    pltpu.make_async_copy(
                        abuf, rows_out.at[pl.ds(0, _P // 4)], sem_ws[par]).wait()
                    pltpu.make_async_copy(
                        kbufs[par],
                        knn_out.at[b, pl.ds(0, 3), pl.ds(0, rpc)],
                        sem_ws[par]).wait()
            nci = ci + 1

            @pl.when(nci < n_chunks)
            def _():
                issue(nci, 1 - par)

            def pt_body(i, _2):
                for q in range(4):
                    p = i * 4 + q
                    r0 = p * K
                    a0 = rowbuf[r0, pl.ds(0, _LANES)]
                    a1 = rowbuf[r0, pl.ds(_LANES, _LANES)]
                    for kk in range(1, K):
                        a0 = jnp.maximum(a0, rowbuf[r0 + kk, pl.ds(0, _LANES)])
                        a1 = jnp.maximum(a1,
                                         rowbuf[r0 + kk, pl.ds(_LANES, _LANES)])
                    a0 = jnp.maximum(a0 + vbuf[i, pl.ds(q * _CH, _LANES)], 0.0)
                    a1 = jnp.maximum(
                        a1 + vbuf[i, pl.ds(q * _CH + _LANES, _LANES)], 0.0)
                    abuf[i, pl.ds(q * _CH, _LANES)] = a0
                    abuf[i, pl.ds(q * _CH + _LANES, _LANES)] = a1
                return 0

            lax.fori_loop(0, _P // 4, pt_body, 0)
            if emit_extras:
                pltpu.async_copy(abuf, rows_out.at[pl.ds(pt0 // 4, _P // 4)],
                                 sem_ws[par])

            iod4 = iot // 4
            iom4 = (iot % 4) * _CH

            def grp_body(gi, _2):
                rows = iod4 + gi * 4
                for c in range(24):
                    vals = plsc.load_gather(abuf, [rows, iom4 + c])
                    cmbuf[c, pl.ds(ci * _P + gi * _LANES, _LANES)] = vals
                return 0

            lax.fori_loop(0, _P // _LANES, grp_body, 0)

            if emit_extras:
                kbuf = kbufs[par]

                kcols = [iot * 0 + (24 + c) for c in range(3)]

                def kgrp_body(g4, _2):
                    for u in range(4):
                        gi = g4 * 4 + u
                        rows = iot + gi * _LANES
                        for c in range(3):
                            vals = plsc.load_gather(rowbuf, [rows, kcols[c]])
                            kbuf[c, pl.ds(gi * _LANES, _LANES)] = vals
                    return 0

                lax.fori_loop(0, rpc // _LANES // 4, kgrp_body, 0)
                flat0 = pl.multiple_of((pt_in_b + ci * _P) * K, rpc)
                pltpu.async_copy(kbuf,
                                 knn_out.at[b, pl.ds(0, 3), pl.ds(flat0, rpc)],
                                 sem_ws[par])

        issue(0, 0)

        def pair_body(i, _):
            compute(2 * i, 0)
            compute(2 * i + 1, 1)
            return 0

        lax.fori_loop(0, n_chunks // 2, pair_body, 0)
        if emit_extras:
            for par in range(2):
                pltpu.make_async_copy(
                    abufs[par], rows_out.at[pl.ds(0, _P // 4)],
                    sem_ws[par]).wait()
                pltpu.make_async_copy(
                    kbufs[par], knn_out.at[b, pl.ds(0, 3), pl.ds(0, rpc)],
                    sem_ws[par]).wait()
        for c8 in range(0, 24, 8):
            pltpu.sync_copy(cmbuf.at[pl.ds(c8, 8)],
                            cm_out.at[b, pl.ds(c8, 8), pl.ds(pt_in_b, pts_w)])

    res = k(table_flat, idx_rows, v_flat)
    if not emit_extras and isinstance(res, (tuple, list)):
        return res[0]
    return res


def _tcc0_body(pos_ref, sp_ref, ab_ref, m1w_ref, m1b_ref, m2w_ref, m2b_ref,
               xe_ref, pf_ref, dwf_ref, dw_ref):
    pos3 = pos_ref[0]
    N = pos3.shape[1]
    scale = sp_ref[:, 0:1]
    phase = sp_ref[:, 1:2]
    rep = jnp.concatenate(
        [jnp.broadcast_to(pos3[c:c + 1, :], (24, N)) for c in range(3)], axis=0)
    xe_ref[0] = jnp.sin(rep * scale + phase)

    h = _relu(_dot(m1w_ref[...], pos3) + m1b_ref[...])
    pf_ref[0] = _dot(m2w_ref[...], h) + m2b_ref[...]

    dist = jnp.sqrt(jnp.sum(pos3 * pos3, axis=0, keepdims=True))
    a_ = ab_ref[:, 0:1]
    b_ = ab_ref[:, 1:2]
    dwr = jax.nn.sigmoid(-(a_ * dist) + b_)
    dws = jnp.sum(dwr, axis=1, keepdims=True)
    dws = dws + (dws == 0).astype(_F32) + 1e-6
    dist_w = dwr / dws * float(N)
    dwf_ref[0] = dist_w
    dw_ref[0] = dist_w[:, :N // 8]


def _tc_c0(pos, sp, ab, m1w, m1b, m2w, m2b):
    B, _, N = pos.shape
    full = lambda a: pl.BlockSpec(a.shape, lambda b: (0,) * a.ndim)
    return pl.pallas_call(
        _tcc0_body,
        grid=(B,),
        in_specs=[pl.BlockSpec((1, 3, N), lambda b: (b, 0, 0)),
                  full(sp), full(ab), full(m1w), full(m1b), full(m2w),
                  full(m2b)],
        out_specs=[
            pl.BlockSpec((1, 72, N), lambda b: (b, 0, 0)),
            pl.BlockSpec((1, 64, N), lambda b: (b, 0, 0)),
            pl.BlockSpec((1, 1, N), lambda b: (b, 0, 0)),
            pl.BlockSpec((1, 1, N // 8), lambda b: (b, 0, 0)),
        ],
        out_shape=[
            jax.ShapeDtypeStruct((B, 72, N), _F32),
            jax.ShapeDtypeStruct((B, 64, N), _F32),
            jax.ShapeDtypeStruct((B, 1, N), _F32),
            jax.ShapeDtypeStruct((B, 1, N // 8), _F32),
        ],
    )(pos, sp, ab, m1w, m1b, m2w, m2b)


def _tcc1_body(*refs):
    (y_ref, xe_ref, pf_ref, dwf_ref,
     l0w_ref, l0b_ref, l1w_ref, l1b_ref, l2w_ref, l2b_ref,
     l3w_ref, l3b_ref, l4w_ref, l4b_ref,
     c1w_ref, c1b_ref,
     p1w_ref, p1b_ref, p2w_ref, p2b_ref, p3w_ref, p3b_ref,
     p4w_ref, p4b_ref, p5w_ref, p5b_ref,
     c2w_ref, c2b_ref, c3w_ref, c3b_ref,
     out_ref, yout_ref) = refs

    y24 = y_ref[0]
    xe = xe_ref[0]
    pos_f = pf_ref[0]
    dist_w = dwf_ref[0]
    N = y24.shape[1]

    x = jnp.concatenate([pos_f, y24], axis=0)
    x = _relu(_dot(l0w_ref[...], x) + l0b_ref[...])
    x = _relu(_dot(l1w_ref[...], x) + l1b_ref[...])
    x = _relu(_dot(l2w_ref[...], x) + l2b_ref[...])
    x = jnp.concatenate([x, pos_f, y24], axis=0)
    x = _relu(_dot(l3w_ref[...], x) + l3b_ref[...])
    yy = _dot(l4w_ref[...], x) + l4b_ref[...]

    ycat = jnp.concatenate([y24, yy, xe], axis=0)
    yt = _relu(_dot(c1w_ref[...], ycat) + c1b_ref[...])
    yout_ref[0] = yt[:, :N // 8]

    def pile(xx, w, bb, n):
        w_in = xx.shape[1]
        C = xx.shape[0]
        g = jnp.mean(xx * dist_w[:, :w_in], axis=1, keepdims=True)
        corr = _dot(w[:, C:], g) + bb
        return _relu(_dot(w[:, :C], xx) + corr)[:, :n]

    y1 = pile(yt, p1w_ref[...], p1b_ref[...], N // 2)
    y2 = pile(y1, p2w_ref[...], p2b_ref[...], N // 4) + y1[:, :N // 4]
    y3 = pile(y2, p3w_ref[...], p3b_ref[...], N // 4)
    y4 = pile(y3, p4w_ref[...], p4b_ref[...], N // 8) + y3[:, :N // 8]
    y5 = pile(y4, p5w_ref[...], p5b_ref[...], N // 8)
    o = _relu(_dot(c2w_ref[...], y5) + c2b_ref[...])
    out_ref[0] = _relu(_dot(c3w_ref[...], o) + c3b_ref[...])


def _tc_c1(ycm, xe, pos_f, dwf, weights):
    B, _, N = ycm.shape
    full = lambda a: pl.BlockSpec(a.shape, lambda b: (0,) * a.ndim)
    in_specs = [
        pl.BlockSpec((1, 24, N), lambda b: (b, 0, 0)),
        pl.BlockSpec((1, 72, N), lambda b: (b, 0, 0)),
        pl.BlockSpec((1, 64, N), lambda b: (b, 0, 0)),
        pl.BlockSpec((1, 1, N), lambda b: (b, 0, 0)),
    ] + [full(w) for w in weights]
    out_specs = [
        pl.BlockSpec((1, 128, N // 8), lambda b: (b, 0, 0)),
        pl.BlockSpec((1, 128, N // 8), lambda b: (b, 0, 0)),
    ]
    out_shape = [
        jax.ShapeDtypeStruct((B, 128, N // 8), _F32),
        jax.ShapeDtypeStruct((B, 128, N // 8), _F32),
    ]
    return pl.pallas_call(
        _tcc1_body,
        grid=(B,),
        in_specs=in_specs,
        out_specs=out_specs,
        out_shape=out_shape,
    )(ycm, xe, pos_f, dwf, *weights)


def kernel(pos, num_pcl, knn_idx, params):
    p = params
    B, Cin, N = pos.shape
    K = knn_idx.shape[-1]

    posT12 = jnp.swapaxes(pos, 1, 2).reshape(B, N // 4, 4 * Cin)
    idx_rows = knn_idx.astype(jnp.int32).reshape(B * N * K // 128, 128)

    out_dim, alpha_c, beta_c = 72, 1000.0, 100.0
    feat_dim = out_dim // (Cin * 2)
    fr = np.arange(feat_dim, dtype=np.float32)
    dim_embed = np.power(alpha_c, fr / feat_dim)
    scale_np = np.zeros((out_dim,), np.float32)
    phase_np = np.zeros((out_dim,), np.float32)
    for c in range(Cin):
        for f in range(feat_dim):
            scale_np[c * 24 + 2 * f] = beta_c / dim_embed[f]
            scale_np[c * 24 + 2 * f + 1] = beta_c / dim_embed[f]
            phase_np[c * 24 + 2 * f + 1] = np.pi / 2
    sp = jnp.asarray(np.stack([scale_np, phase_np], axis=1))
    ab = jnp.stack([p['alpha'].astype(_F32),
                    p['beta'].astype(_F32)]).reshape(1, 2)

    x_embed, pos_f, dwf, dist_w = _tc_c0(
        pos, sp, ab, p['mlp1_W1'], p['mlp1_b1'].reshape(-1, 1),
        p['mlp1_W2'], p['mlp1_b2'].reshape(-1, 1))

    W1 = p['enc_W1']
    W1d, W1c = W1[:, :Cin], W1[:, Cin:]
    wt1 = jnp.zeros((Cin, _CH), _F32).at[:, :24].set(W1d.T)
    wt1 = wt1.at[:, 24:24 + Cin].set(jnp.eye(Cin, dtype=_F32))
    wv1 = jnp.zeros((Cin, _CH), _F32).at[:, :24].set((W1c - W1d).T)
    bv1 = jnp.zeros((_CH,), _F32).at[:24].set(p['enc_b1'])

    table1, v1rows = _tc_ab(posT12, _blockdiag4(wt1), _blockdiag4(wv1),
                            jnp.tile(bv1, 4).reshape(1, 128))
    a1rows, aa, knn_flat = _sc_edge_gather(
        table1.reshape(B * N, _CH), idx_rows,
        v1rows.reshape(B * N // 4, 128), B, N, K, emit_extras=True)

    W2 = p['enc_W2']
    W2d, W2c = W2[:, :24], W2[:, 24:]
    wt2 = jnp.zeros((_CH, _CH), _F32).at[:24, :24].set(W2d.T)
    wv2 = jnp.zeros((_CH, _CH), _F32).at[:24, :24].set((W2c - W2d).T)
    bv2 = jnp.zeros((_CH,), _F32).at[:24].set(p['enc_b2'])

    table2, v2rows = _tc_ab(a1rows.reshape(B, N // 4, 128),
                            _blockdiag4(wt2), _blockdiag4(wv2),
                            jnp.tile(bv2, 4).reshape(1, 128))
    gg = _sc_edge_gather(
        table2.reshape(B * N, _CH), idx_rows,
        v2rows.reshape(B * N // 4, 128), B, N, K, emit_extras=False)

    weights = [
        p['lin0_W'], p['lin0_b'].reshape(-1, 1),
        p['lin1_W'], p['lin1_b'].reshape(-1, 1),
        p['lin2_W'], p['lin2_b'].reshape(-1, 1),
        p['lin3_W'], p['lin3_b'].reshape(-1, 1),
        p['lin4_W'], p['lin4_b'].reshape(-1, 1),
        p['conv1_W'], p['conv1_b'].reshape(-1, 1),
        p['p1_W'], p['p1_b'].reshape(-1, 1),
        p['p2_W'], p['p2_b'].reshape(-1, 1),
        p['p3_W'], p['p3_b'].reshape(-1, 1),
        p['p4_W'], p['p4_b'].reshape(-1, 1),
        p['p5_W'], p['p5_b'].reshape(-1, 1),
        p['conv2_W'], p['conv2_b'].reshape(-1, 1),
        p['conv3_W'], p['conv3_b'].reshape(-1, 1),
    ]
    out, y_out = _tc_c1(gg, x_embed, pos_f, dwf, weights)
    knn_pos_out = knn_flat.reshape(B, Cin, N, K)
    return (out, dist_w, y_out, aa, gg, knn_pos_out, x_embed)


# speedup vs baseline: 1.0095x; 1.0095x over previous
import functools

import numpy as np
import jax
import jax.numpy as jnp
from jax import lax
from jax.experimental import pallas as pl
from jax.experimental.pallas import tpu as pltpu
from jax.experimental.pallas import tpu_sc as plsc

_F32 = jnp.float32

_NC, _NS, _LANES = 2, 16, 16
_NW = _NC * _NS
_CH = 32
_P = 64


def _relu(x):
    return jnp.maximum(x, 0.0)


def _dot(a, b):
    return jnp.dot(a, b, preferred_element_type=_F32)


def _tcab_body(x_ref, wt_ref, wv_ref, bv_ref, table_ref, v_ref):
    x = x_ref[0]
    table_ref[0] = _dot(x, wt_ref[...])
    v_ref[0] = _dot(x, wv_ref[...]) + bv_ref[...]


def _tc_ab(x, wt, wv, bv):
    B, R, CIN4 = x.shape
    return pl.pallas_call(
        _tcab_body,
        grid=(B,),
        in_specs=[
            pl.BlockSpec((1, R, CIN4), lambda b: (b, 0, 0)),
            pl.BlockSpec((CIN4, 128), lambda b: (0, 0)),
            pl.BlockSpec((CIN4, 128), lambda b: (0, 0)),
            pl.BlockSpec((1, 128), lambda b: (0, 0)),
        ],
        out_specs=[
            pl.BlockSpec((1, R, 128), lambda b: (b, 0, 0)),
            pl.BlockSpec((1, R, 128), lambda b: (b, 0, 0)),
        ],
        out_shape=[
            jax.ShapeDtypeStruct((B, R, 128), _F32),
            jax.ShapeDtypeStruct((B, R, 128), _F32),
        ],
    )(x, wt, wv, bv)


def _blockdiag4(w):
    cin = w.shape[0]
    out = jnp.zeros((4 * cin, 128), _F32)
    for q in range(4):
        out = out.at[q * cin:(q + 1) * cin, q * _CH:(q + 1) * _CH].set(w)
    return out


def _sc_edge_gather(table_flat, idx_rows, v_flat, B, N, K, emit_extras):
    pts_w = B * N // _NW
    n_chunks = pts_w // _P
    rpc = _P * K
    ng = rpc // 128
    per_b_w = _NW // B
    mesh = plsc.VectorSubcoreMesh(core_axis_name="c", subcore_axis_name="s")

    out_type = []
    if emit_extras:
        out_type.append(jax.ShapeDtypeStruct((B * N // 4, 128), _F32))
    out_type.append(jax.ShapeDtypeStruct((B, 24, N), _F32))
    if emit_extras:
        out_type.append(jax.ShapeDtypeStruct((B, 3, N * K), _F32))

    idx_rows_w = pts_w * K // 128
    scratch_types = [
        pltpu.VMEM((idx_rows_w, 128), jnp.int32),
        pltpu.VMEM((rpc, _CH), _F32),
        pltpu.VMEM((rpc, _CH), _F32),
        pltpu.VMEM((_P // 4, 128), _F32),
        pltpu.VMEM((_P // 4, 128), _F32),
        pltpu.VMEM((_P // 4, 128), _F32),
        pltpu.VMEM((_P // 4, 128), _F32),
        pltpu.VMEM((24, pts_w), _F32),
        pltpu.SemaphoreType.DMA,
        pltpu.SemaphoreType.DMA,
        pltpu.SemaphoreType.DMA,
        pltpu.SemaphoreType.DMA,
    ]
    if emit_extras:
        scratch_types.append(pltpu.VMEM((3, rpc), _F32))
        scratch_types.append(pltpu.VMEM((3, rpc), _F32))

    @functools.partial(
        pl.kernel, out_type=tuple(out_type), mesh=mesh,
        scratch_types=scratch_types,
        compiler_params=pltpu.CompilerParams(use_tc_tiling_on_sc=False,
                                             needs_layout_passes=False))
    def k(table, idxg, vrows, *rest):
        if emit_extras:
            rows_out, cm_out, knn_out = rest[0], rest[1], rest[2]
            (idxall, rowbuf_a, rowbuf_b, vbuf_a, vbuf_b, abuf_a, abuf_b,
             cmbuf, sem, sem_v, sem_wa, sem_wb, kbuf_a, kbuf_b) = rest[3:]
            kbufs = (kbuf_a, kbuf_b)
        else:
            cm_out = rest[0]
            (idxall, rowbuf_a, rowbuf_b, vbuf_a, vbuf_b, abuf_a, abuf_b,
             cmbuf, sem, sem_v, sem_wa, sem_wb) = rest[1:]

        rowbufs = (rowbuf_a, rowbuf_b)
        vbufs = (vbuf_a, vbuf_b)
        abufs = (abuf_a, abuf_b)
        sem_ws = (sem_wa, sem_wb)
        wid = lax.axis_index("s") * _NC + lax.axis_index("c")
        base_pt = wid * pts_w
        b = wid // per_b_w
        pt_in_b = base_pt - b * N
        iot = lax.iota(jnp.int32, _LANES)

        irow0 = pl.multiple_of(base_pt * K // 128, 8)
        pltpu.sync_copy(idxg.at[pl.ds(irow0, idx_rows_w)], idxall)

        def issue(ci, par):
            pt0 = pl.multiple_of(base_pt + ci * _P, _P)
            for j in range(ng):
                pltpu.async_copy(table.at[idxall.at[ci * ng + j]],
                                 rowbufs[par].at[pl.ds(j * 128, 128)], sem)
            pltpu.async_copy(vrows.at[pl.ds(pt0 // 4, _P // 4)], vbufs[par],
                             sem_v)

        def compute(ci, par):
            rowbuf, vbuf, abuf = rowbufs[par], vbufs[par], abufs[par]
            pt0 = pl.multiple_of(base_pt + ci * _P, _P)
            pltpu.make_async_copy(table.at[pl.ds(0, rpc)], rowbuf, sem).wait()
            pltpu.make_async_copy(vrows.at[pl.ds(0, _P // 4)], vbuf,
                                  sem_v).wait()
            if emit_extras:
                @pl.when(ci >= 2)
                def _():
                    pltpu.make_async_copy(
                        abuf, rows_out.at[pl.ds(0, _P // 4)], sem_ws[par]).wait()
                    pltpu.make_async_copy(
                        kbufs[par],
                        knn_out.at[b, pl.ds(0, 3), pl.ds(0, rpc)],
                        sem_ws[par]).wait()
            nci = ci + 1

            @pl.when(nci < n_chunks)
            def _():
                issue(nci, 1 - par)

            def pt_body(i, _2):
                for q in range(4):
                    p = i * 4 + q
                    r0 = p * K
                    a0 = rowbuf[r0, pl.ds(0, _LANES)]
                    a1 = rowbuf[r0, pl.ds(_LANES, _LANES)]
                    for kk in range(1, K):
                        a0 = jnp.maximum(a0, rowbuf[r0 + kk, pl.ds(0, _LANES)])
                        a1 = jnp.maximum(a1,
                                         rowbuf[r0 + kk, pl.ds(_LANES, _LANES)])
                    a0 = jnp.maximum(a0 + vbuf[i, pl.ds(q * _CH, _LANES)], 0.0)
                    a1 = jnp.maximum(
                        a1 + vbuf[i, pl.ds(q * _CH + _LANES, _LANES)], 0.0)
                    abuf[i, pl.ds(q * _CH, _LANES)] = a0
                    abuf[i, pl.ds(q * _CH + _LANES, _LANES)] = a1
                return 0

            lax.fori_loop(0, _P // 4, pt_body, 0)
            if emit_extras:
                pltpu.async_copy(abuf, rows_out.at[pl.ds(pt0 // 4, _P // 4)],
                                 sem_ws[par])

            iod4 = iot // 4
            iom4 = (iot % 4) * _CH

            def grp_body(gi, _2):
                rows = iod4 + gi * 4
                for c in range(24):
                    vals = plsc.load_gather(abuf, [rows, iom4 + c])
                    cmbuf[c, pl.ds(ci * _P + gi * _LANES, _LANES)] = vals
                return 0

            lax.fori_loop(0, _P // _LANES, grp_body, 0)

            if emit_extras:
                kbuf = kbufs[par]

                kcols = [iot * 0 + (24 + c) for c in range(3)]

                def kgrp_body(g4, _2):
                    for u in range(4):
                        gi = g4 * 4 + u
                        rows = iot + gi * _LANES
                        for c in range(3):
                            vals = plsc.load_gather(rowbuf, [rows, kcols[c]])
                            kbuf[c, pl.ds(gi * _LANES, _LANES)] = vals
                    return 0

                lax.fori_loop(0, rpc // _LANES // 4, kgrp_body, 0)
                flat0 = pl.multiple_of((pt_in_b + ci * _P) * K, rpc)
                pltpu.async_copy(kbuf,
                                 knn_out.at[b, pl.ds(0, 3), pl.ds(flat0, rpc)],
                                 sem_ws[par])

        issue(0, 0)

        def pair_body(i, _):
            compute(2 * i, 0)
            compute(2 * i + 1, 1)
            return 0

        lax.fori_loop(0, n_chunks // 2, pair_body, 0)
        if emit_extras:
            for par in range(2):
                pltpu.make_async_copy(
                    abufs[par], rows_out.at[pl.ds(0, _P // 4)],
                    sem_ws[par]).wait()
                pltpu.make_async_copy(
                    kbufs[par], knn_out.at[b, pl.ds(0, 3), pl.ds(0, rpc)],
                    sem_ws[par]).wait()
        for c8 in range(0, 24, 8):
            pltpu.sync_copy(cmbuf.at[pl.ds(c8, 8)],
                            cm_out.at[b, pl.ds(c8, 8), pl.ds(pt_in_b, pts_w)])

    res = k(table_flat, idx_rows, v_flat)
    if not emit_extras and isinstance(res, (tuple, list)):
        return res[0]
    return res


def _tcc1_body(*refs):
    (pos_ref, y_ref, xe_ref, ab_ref,
     m1w_ref, m1b_ref, m2w_ref, m2b_ref,
     l0w_ref, l0b_ref, l1w_ref, l1b_ref, l2w_ref, l2b_ref,
     l3w_ref, l3b_ref, l4w_ref, l4b_ref,
     c1w_ref, c1b_ref,
     p1w_ref, p1b_ref, p2w_ref, p2b_ref, p3w_ref, p3b_ref,
     p4w_ref, p4b_ref, p5w_ref, p5b_ref,
     c2w_ref, c2b_ref, c3w_ref, c3b_ref,
     out_ref, dw_ref, yout_ref) = refs

    pos3 = pos_ref[0]
    y24 = y_ref[0]
    xe = xe_ref[0]
    N = y24.shape[1]

    h = _relu(_dot(m1w_ref[...], pos3) + m1b_ref[...])
    pos_f = _dot(m2w_ref[...], h) + m2b_ref[...]

    dist = jnp.sqrt(jnp.sum(pos3 * pos3, axis=0, keepdims=True))
    a_ = ab_ref[:, 0:1]
    b_ = ab_ref[:, 1:2]
    dwr = jax.nn.sigmoid(-(a_ * dist) + b_)
    dws = jnp.sum(dwr, axis=1, keepdims=True)
    dws = dws + (dws == 0).astype(_F32) + 1e-6
    dist_w = dwr / dws * float(N)
    dw_ref[0] = dist_w[:, :N // 8]

    x = jnp.concatenate([pos_f, y24], axis=0)
    x = _relu(_dot(l0w_ref[...], x) + l0b_ref[...])
    x = _relu(_dot(l1w_ref[...], x) + l1b_ref[...])
    x = _relu(_dot(l2w_ref[...], x) + l2b_ref[...])
    x = jnp.concatenate([x, pos_f, y24], axis=0)
    x = _relu(_dot(l3w_ref[...], x) + l3b_ref[...])
    yy = _dot(l4w_ref[...], x) + l4b_ref[...]

    ycat = jnp.concatenate([y24, yy, xe], axis=0)
    yt = _relu(_dot(c1w_ref[...], ycat) + c1b_ref[...])
    yout_ref[0] = yt[:, :N // 8]

    def pile(xx, w, bb, n):
        w_in = xx.shape[1]
        C = xx.shape[0]
        g = jnp.mean(xx * dist_w[:, :w_in], axis=1, keepdims=True)
        corr = _dot(w[:, C:], g) + bb
        return _relu(_dot(w[:, :C], xx) + corr)[:, :n]

    y1 = pile(yt, p1w_ref[...], p1b_ref[...], N // 2)
    y2 = pile(y1, p2w_ref[...], p2b_ref[...], N // 4) + y1[:, :N // 4]
    y3 = pile(y2, p3w_ref[...], p3b_ref[...], N // 4)
    y4 = pile(y3, p4w_ref[...], p4b_ref[...], N // 8) + y3[:, :N // 8]
    y5 = pile(y4, p5w_ref[...], p5b_ref[...], N // 8)
    o = _relu(_dot(c2w_ref[...], y5) + c2b_ref[...])
    out_ref[0] = _relu(_dot(c3w_ref[...], o) + c3b_ref[...])


def _tc_c1(pos, ycm, xe, ab, weights):
    B, _, N = pos.shape
    full = lambda a: pl.BlockSpec(a.shape, lambda b: (0,) * a.ndim)
    in_specs = [
        pl.BlockSpec((1, 3, N), lambda b: (b, 0, 0)),
        pl.BlockSpec((1, 24, N), lambda b: (b, 0, 0)),
        pl.BlockSpec((1, 72, N), lambda b: (b, 0, 0)),
        full(ab),
    ] + [full(w) for w in weights]
    out_specs = [
        pl.BlockSpec((1, 128, N // 8), lambda b: (b, 0, 0)),
        pl.BlockSpec((1, 1, N // 8), lambda b: (b, 0, 0)),
        pl.BlockSpec((1, 128, N // 8), lambda b: (b, 0, 0)),
    ]
    out_shape = [
        jax.ShapeDtypeStruct((B, 128, N // 8), _F32),
        jax.ShapeDtypeStruct((B, 1, N // 8), _F32),
        jax.ShapeDtypeStruct((B, 128, N // 8), _F32),
    ]
    return pl.pallas_call(
        _tcc1_body,
        grid=(B,),
        in_specs=in_specs,
        out_specs=out_specs,
        out_shape=out_shape,
    )(pos, ycm, xe, ab, *weights)


def kernel(pos, num_pcl, knn_idx, params):
    p = params
    B, Cin, N = pos.shape
    K = knn_idx.shape[-1]

    posT12 = jnp.swapaxes(pos, 1, 2).reshape(B, N // 4, 4 * Cin)
    idx32 = knn_idx.astype(jnp.int32)
    idx_g = (idx32 + (jnp.arange(B, dtype=jnp.int32) * N)[:, None, None])
    idx_rows = idx_g.reshape(B * N * K // 128, 128)

    out_dim, alpha_c, beta_c = 72, 1000.0, 100.0
    feat_dim = out_dim // (Cin * 2)
    fr = np.arange(feat_dim, dtype=np.float32)
    dim_embed = np.power(alpha_c, fr / feat_dim)
    scale_np = np.zeros((out_dim,), np.float32)
    phase_np = np.zeros((out_dim,), np.float32)
    for c in range(Cin):
        for f in range(feat_dim):
            scale_np[c * 24 + 2 * f] = beta_c / dim_embed[f]
            scale_np[c * 24 + 2 * f + 1] = beta_c / dim_embed[f]
            phase_np[c * 24 + 2 * f + 1] = np.pi / 2
    ab = jnp.stack([p['alpha'].astype(_F32),
                    p['beta'].astype(_F32)]).reshape(1, 2)
    x_embed = jnp.sin(
        jnp.repeat(pos, feat_dim * 2, axis=1)
        * jnp.asarray(scale_np)[None, :, None]
        + jnp.asarray(phase_np)[None, :, None])

    W1 = p['enc_W1']
    W1d, W1c = W1[:, :Cin], W1[:, Cin:]
    wt1 = jnp.zeros((Cin, _CH), _F32).at[:, :24].set(W1d.T)
    wt1 = wt1.at[:, 24:24 + Cin].set(jnp.eye(Cin, dtype=_F32))
    wv1 = jnp.zeros((Cin, _CH), _F32).at[:, :24].set((W1c - W1d).T)
    bv1 = jnp.zeros((_CH,), _F32).at[:24].set(p['enc_b1'])

    table1, v1rows = _tc_ab(posT12, _blockdiag4(wt1), _blockdiag4(wv1),
                            jnp.tile(bv1, 4).reshape(1, 128))
    a1rows, aa, knn_flat = _sc_edge_gather(
        table1.reshape(B * N, _CH), idx_rows,
        v1rows.reshape(B * N // 4, 128), B, N, K, emit_extras=True)

    W2 = p['enc_W2']
    W2d, W2c = W2[:, :24], W2[:, 24:]
    wt2 = jnp.zeros((_CH, _CH), _F32).at[:24, :24].set(W2d.T)
    wv2 = jnp.zeros((_CH, _CH), _F32).at[:24, :24].set((W2c - W2d).T)
    bv2 = jnp.zeros((_CH,), _F32).at[:24].set(p['enc_b2'])

    table2, v2rows = _tc_ab(a1rows.reshape(B, N // 4, 128),
                            _blockdiag4(wt2), _blockdiag4(wv2),
                            jnp.tile(bv2, 4).reshape(1, 128))
    gg = _sc_edge_gather(
        table2.reshape(B * N, _CH), idx_rows,
        v2rows.reshape(B * N // 4, 128), B, N, K, emit_extras=False)

    weights = [
        p['mlp1_W1'], p['mlp1_b1'].reshape(-1, 1),
        p['mlp1_W2'], p['mlp1_b2'].reshape(-1, 1),
        p['lin0_W'], p['lin0_b'].reshape(-1, 1),
        p['lin1_W'], p['lin1_b'].reshape(-1, 1),
        p['lin2_W'], p['lin2_b'].reshape(-1, 1),
        p['lin3_W'], p['lin3_b'].reshape(-1, 1),
        p['lin4_W'], p['lin4_b'].reshape(-1, 1),
        p['conv1_W'], p['conv1_b'].reshape(-1, 1),
        p['p1_W'], p['p1_b'].reshape(-1, 1),
        p['p2_W'], p['p2_b'].reshape(-1, 1),
        p['p3_W'], p['p3_b'].reshape(-1, 1),
        p['p4_W'], p['p4_b'].reshape(-1, 1),
        p['p5_W'], p['p5_b'].reshape(-1, 1),
        p['conv2_W'], p['conv2_b'].reshape(-1, 1),
        p['conv3_W'], p['conv3_b'].reshape(-1, 1),
    ]
    out, dist_w, y_out = _tc_c1(pos, gg, x_embed, ab, weights)
    knn_pos_out = knn_flat.reshape(B, Cin, N, K)
    return (out, dist_w, y_out, aa, gg, knn_pos_out, x_embed)


# speedup vs baseline: 1.0278x; 1.0181x over previous
import functools

import numpy as np
import jax
import jax.numpy as jnp
from jax import lax
from jax.experimental import pallas as pl
from jax.experimental.pallas import tpu as pltpu
from jax.experimental.pallas import tpu_sc as plsc

_F32 = jnp.float32

_NC, _NS, _LANES = 2, 16, 16
_NW = _NC * _NS
_CH = 32
_P = 64


def _relu(x):
    return jnp.maximum(x, 0.0)


def _dot(a, b):
    return jnp.dot(a, b, preferred_element_type=_F32)


def _tcab_body(x_ref, wt_ref, wv_ref, bv_ref, table_ref, v_ref):
    x = x_ref[0]
    table_ref[0] = _dot(x, wt_ref[...])
    v_ref[0] = _dot(x, wv_ref[...]) + bv_ref[...]


def _tc_ab(x, wt, wv, bv):
    B, R, CIN4 = x.shape
    return pl.pallas_call(
        _tcab_body,
        grid=(B,),
        in_specs=[
            pl.BlockSpec((1, R, CIN4), lambda b: (b, 0, 0)),
            pl.BlockSpec((CIN4, 128), lambda b: (0, 0)),
            pl.BlockSpec((CIN4, 128), lambda b: (0, 0)),
            pl.BlockSpec((1, 128), lambda b: (0, 0)),
        ],
        out_specs=[
            pl.BlockSpec((1, R, 128), lambda b: (b, 0, 0)),
            pl.BlockSpec((1, R, 128), lambda b: (b, 0, 0)),
        ],
        out_shape=[
            jax.ShapeDtypeStruct((B, R, 128), _F32),
            jax.ShapeDtypeStruct((B, R, 128), _F32),
        ],
    )(x, wt, wv, bv)


def _blockdiag4(w):
    cin = w.shape[0]
    out = jnp.zeros((4 * cin, 128), _F32)
    for q in range(4):
        out = out.at[q * cin:(q + 1) * cin, q * _CH:(q + 1) * _CH].set(w)
    return out


def _sc_edge_gather(table_flat, idx_rows, v_flat, B, N, K, emit_extras):
    pts_w = B * N // _NW
    n_chunks = pts_w // _P
    rpc = _P * K
    ng = rpc // 128
    per_b_w = _NW // B
    mesh = plsc.VectorSubcoreMesh(core_axis_name="c", subcore_axis_name="s")

    out_type = []
    if emit_extras:
        out_type.append(jax.ShapeDtypeStruct((B * N // 4, 128), _F32))
    out_type.append(jax.ShapeDtypeStruct((B, 24, N), _F32))
    if emit_extras:
        out_type.append(jax.ShapeDtypeStruct((B, 3, N * K), _F32))

    idx_rows_w = pts_w * K // 128
    scratch_types = [
        pltpu.VMEM((idx_rows_w, 128), jnp.int32),
        pltpu.VMEM((rpc, _CH), _F32),
        pltpu.VMEM((rpc, _CH), _F32),
        pltpu.VMEM((_P // 4, 128), _F32),
        pltpu.VMEM((_P // 4, 128), _F32),
        pltpu.VMEM((_P // 4, 128), _F32),
        pltpu.VMEM((_P // 4, 128), _F32),
        pltpu.VMEM((24, pts_w), _F32),
        pltpu.SemaphoreType.DMA,
        pltpu.SemaphoreType.DMA,
        pltpu.SemaphoreType.DMA,
        pltpu.SemaphoreType.DMA,
    ]
    if emit_extras:
        scratch_types.append(pltpu.VMEM((3, rpc), _F32))
        scratch_types.append(pltpu.VMEM((3, rpc), _F32))

    @functools.partial(
        pl.kernel, out_type=tuple(out_type), mesh=mesh,
        scratch_types=scratch_types,
        compiler_params=pltpu.CompilerParams(use_tc_tiling_on_sc=False,
                                             needs_layout_passes=False))
    def k(table, idxg, vrows, *rest):
        if emit_extras:
            rows_out, cm_out, knn_out = rest[0], rest[1], rest[2]
            (idxall, rowbuf_a, rowbuf_b, vbuf_a, vbuf_b, abuf_a, abuf_b,
             cmbuf, sem, sem_v, sem_wa, sem_wb, kbuf_a, kbuf_b) = rest[3:]
            kbufs = (kbuf_a, kbuf_b)
        else:
            cm_out = rest[0]
            (idxall, rowbuf_a, rowbuf_b, vbuf_a, vbuf_b, abuf_a, abuf_b,
             cmbuf, sem, sem_v, sem_wa, sem_wb) = rest[1:]

        rowbufs = (rowbuf_a, rowbuf_b)
        vbufs = (vbuf_a, vbuf_b)
        abufs = (abuf_a, abuf_b)
        sem_ws = (sem_wa, sem_wb)
        wid = lax.axis_index("s") * _NC + lax.axis_index("c")
        base_pt = wid * pts_w
        b = wid // per_b_w
        pt_in_b = base_pt - b * N
        iot = lax.iota(jnp.int32, _LANES)

        irow0 = pl.multiple_of(base_pt * K // 128, 8)
        pltpu.sync_copy(idxg.at[pl.ds(irow0, idx_rows_w)], idxall)

        def issue(ci, par):
            pt0 = pl.multiple_of(base_pt + ci * _P, _P)
            for j in range(ng):
                pltpu.async_copy(table.at[idxall.at[ci * ng + j]],
                                 rowbufs[par].at[pl.ds(j * 128, 128)], sem)
            pltpu.async_copy(vrows.at[pl.ds(pt0 // 4, _P // 4)], vbufs[par],
                             sem_v)

        def compute(ci, par):
            rowbuf, vbuf, abuf = rowbufs[par], vbufs[par], abufs[par]
            pt0 = pl.multiple_of(base_pt + ci * _P, _P)
            pltpu.make_async_copy(table.at[pl.ds(0, rpc)], rowbuf, sem).wait()
            pltpu.make_async_copy(vrows.at[pl.ds(0, _P // 4)], vbuf,
                                  sem_v).wait()
            if emit_extras:
                @pl.when(ci >= 2)
                def _():
                    pltpu.make_async_copy(
                        abuf, rows_out.at[pl.ds(0, _P // 4)], sem_ws[par]).wait()
                    pltpu.make_async_copy(
                        kbufs[par],
                        knn_out.at[b, pl.ds(0, 3), pl.ds(0, rpc)],
                        sem_ws[par]).wait()
            nci = ci + 1

            @pl.when(nci < n_chunks)
            def _():
                issue(nci, 1 - par)

            def pt_body(i, _2):
                for q in range(4):
                    p = i * 4 + q
                    r0 = p * K
                    a0 = rowbuf[r0, pl.ds(0, _LANES)]
                    a1 = rowbuf[r0, pl.ds(_LANES, _LANES)]
                    for kk in range(1, K):
                        a0 = jnp.maximum(a0, rowbuf[r0 + kk, pl.ds(0, _LANES)])
                        a1 = jnp.maximum(a1,
                                         rowbuf[r0 + kk, pl.ds(_LANES, _LANES)])
                    a0 = jnp.maximum(a0 + vbuf[i, pl.ds(q * _CH, _LANES)], 0.0)
                    a1 = jnp.maximum(
                        a1 + vbuf[i, pl.ds(q * _CH + _LANES, _LANES)], 0.0)
                    abuf[i, pl.ds(q * _CH, _LANES)] = a0
                    abuf[i, pl.ds(q * _CH + _LANES, _LANES)] = a1
                return 0

            lax.fori_loop(0, _P // 4, pt_body, 0)
            if emit_extras:
                pltpu.async_copy(abuf, rows_out.at[pl.ds(pt0 // 4, _P // 4)],
                                 sem_ws[par])

            iod4 = iot // 4
            iom4 = (iot % 4) * _CH

            def grp_body(gi, _2):
                rows = iod4 + gi * 4
                for c in range(24):
                    vals = plsc.load_gather(abuf, [rows, iom4 + c])
                    cmbuf[c, pl.ds(ci * _P + gi * _LANES, _LANES)] = vals
                return 0

            lax.fori_loop(0, _P // _LANES, grp_body, 0)

            if emit_extras:
                kbuf = kbufs[par]

                kcols = [iot * 0 + (24 + c) for c in range(3)]

                def kgrp_body(g4, _2):
                    for u in range(4):
                        gi = g4 * 4 + u
                        rows = iot + gi * _LANES
                        for c in range(3):
                            vals = plsc.load_gather(rowbuf, [rows, kcols[c]])
                            kbuf[c, pl.ds(gi * _LANES, _LANES)] = vals
                    return 0

                lax.fori_loop(0, rpc // _LANES // 4, kgrp_body, 0)
                flat0 = pl.multiple_of((pt_in_b + ci * _P) * K, rpc)
                pltpu.async_copy(kbuf,
                                 knn_out.at[b, pl.ds(0, 3), pl.ds(flat0, rpc)],
                                 sem_ws[par])

        issue(0, 0)

        def pair_body(i, _):
            compute(2 * i, 0)
            compute(2 * i + 1, 1)
            return 0

        lax.fori_loop(0, n_chunks // 2, pair_body, 0)
        if emit_extras:
            for par in range(2):
                pltpu.make_async_copy(
                    abufs[par], rows_out.at[pl.ds(0, _P // 4)],
                    sem_ws[par]).wait()
                pltpu.make_async_copy(
                    kbufs[par], knn_out.at[b, pl.ds(0, 3), pl.ds(0, rpc)],
                    sem_ws[par]).wait()
        for c8 in range(0, 24, 8):
            pltpu.sync_copy(cmbuf.at[pl.ds(c8, 8)],
                            cm_out.at[b, pl.ds(c8, 8), pl.ds(pt_in_b, pts_w)])

    res = k(table_flat, idx_rows, v_flat)
    if not emit_extras and isinstance(res, (tuple, list)):
        return res[0]
    return res


def _tcc1_body(*refs):
    (pos_ref, y_ref, sp_ref, ab_ref,
     m1w_ref, m1b_ref, m2w_ref, m2b_ref,
     l0w_ref, l0b_ref, l1w_ref, l1b_ref, l2w_ref, l2b_ref,
     l3w_ref, l3b_ref, l4w_ref, l4b_ref,
     c1w_ref, c1b_ref,
     p1w_ref, p1b_ref, p2w_ref, p2b_ref, p3w_ref, p3b_ref,
     p4w_ref, p4b_ref, p5w_ref, p5b_ref,
     c2w_ref, c2b_ref, c3w_ref, c3b_ref,
     out_ref, dw_ref, yout_ref, xe_ref) = refs

    pos3 = pos_ref[0]
    y24 = y_ref[0]
    N = y24.shape[1]
    scale = sp_ref[:, 0:1]
    phase = sp_ref[:, 1:2]
    rep = jnp.concatenate(
        [jnp.broadcast_to(pos3[c:c + 1, :], (24, N)) for c in range(3)], axis=0)
    xe = jnp.sin(rep * scale + phase)
    xe_ref[0] = xe

    h = _relu(_dot(m1w_ref[...], pos3) + m1b_ref[...])
    pos_f = _dot(m2w_ref[...], h) + m2b_ref[...]

    dist = jnp.sqrt(jnp.sum(pos3 * pos3, axis=0, keepdims=True))
    a_ = ab_ref[:, 0:1]
    b_ = ab_ref[:, 1:2]
    dwr = jax.nn.sigmoid(-(a_ * dist) + b_)
    dws = jnp.sum(dwr, axis=1, keepdims=True)
    dws = dws + (dws == 0).astype(_F32) + 1e-6
    dist_w = dwr / dws * float(N)
    dw_ref[0] = dist_w[:, :N // 8]

    x = jnp.concatenate([pos_f, y24], axis=0)
    x = _relu(_dot(l0w_ref[...], x) + l0b_ref[...])
    x = _relu(_dot(l1w_ref[...], x) + l1b_ref[...])
    x = _relu(_dot(l2w_ref[...], x) + l2b_ref[...])
    x = jnp.concatenate([x, pos_f, y24], axis=0)
    x = _relu(_dot(l3w_ref[...], x) + l3b_ref[...])
    yy = _dot(l4w_ref[...], x) + l4b_ref[...]

    ycat = jnp.concatenate([y24, yy, xe], axis=0)
    yt = _relu(_dot(c1w_ref[...], ycat) + c1b_ref[...])
    yout_ref[0] = yt[:, :N // 8]

    def pile(xx, w, bb, n):
        w_in = xx.shape[1]
        C = xx.shape[0]
        g = jnp.mean(xx * dist_w[:, :w_in], axis=1, keepdims=True)
        corr = _dot(w[:, C:], g) + bb
        return _relu(_dot(w[:, :C], xx) + corr)[:, :n]

    y1 = pile(yt, p1w_ref[...], p1b_ref[...], N // 2)
    y2 = pile(y1, p2w_ref[...], p2b_ref[...], N // 4) + y1[:, :N // 4]
    y3 = pile(y2, p3w_ref[...], p3b_ref[...], N // 4)
    y4 = pile(y3, p4w_ref[...], p4b_ref[...], N // 8) + y3[:, :N // 8]
    y5 = pile(y4, p5w_ref[...], p5b_ref[...], N // 8)
    o = _relu(_dot(c2w_ref[...], y5) + c2b_ref[...])
    out_ref[0] = _relu(_dot(c3w_ref[...], o) + c3b_ref[...])


def _tc_c1(pos, ycm, sp, ab, weights):
    B, _, N = pos.shape
    full = lambda a: pl.BlockSpec(a.shape, lambda b: (0,) * a.ndim)
    in_specs = [
        pl.BlockSpec((1, 3, N), lambda b: (b, 0, 0)),
        pl.BlockSpec((1, 24, N), lambda b: (b, 0, 0)),
        full(sp), full(ab),
    ] + [full(w) for w in weights]
    out_specs = [
        pl.BlockSpec((1, 128, N // 8), lambda b: (b, 0, 0)),
        pl.BlockSpec((1, 1, N // 8), lambda b: (b, 0, 0)),
        pl.BlockSpec((1, 128, N // 8), lambda b: (b, 0, 0)),
        pl.BlockSpec((1, 72, N), lambda b: (b, 0, 0)),
    ]
    out_shape = [
        jax.ShapeDtypeStruct((B, 128, N // 8), _F32),
        jax.ShapeDtypeStruct((B, 1, N // 8), _F32),
        jax.ShapeDtypeStruct((B, 128, N // 8), _F32),
        jax.ShapeDtypeStruct((B, 72, N), _F32),
    ]
    return pl.pallas_call(
        _tcc1_body,
        grid=(B,),
        in_specs=in_specs,
        out_specs=out_specs,
        out_shape=out_shape,
    )(pos, ycm, sp, ab, *weights)


def kernel(pos, num_pcl, knn_idx, params):
    p = params
    B, Cin, N = pos.shape
    K = knn_idx.shape[-1]

    posT12 = jnp.swapaxes(pos, 1, 2).reshape(B, N // 4, 4 * Cin)
    idx32 = knn_idx.astype(jnp.int32)
    idx_g = (idx32 + (jnp.arange(B, dtype=jnp.int32) * N)[:, None, None])
    idx_rows = idx_g.reshape(B * N * K // 128, 128)

    out_dim, alpha_c, beta_c = 72, 1000.0, 100.0
    feat_dim = out_dim // (Cin * 2)
    fr = np.arange(feat_dim, dtype=np.float32)
    dim_embed = np.power(alpha_c, fr / feat_dim)
    scale_np = np.zeros((out_dim,), np.float32)
    phase_np = np.zeros((out_dim,), np.float32)
    for c in range(Cin):
        for f in range(feat_dim):
            scale_np[c * 24 + 2 * f] = beta_c / dim_embed[f]
            scale_np[c * 24 + 2 * f + 1] = beta_c / dim_embed[f]
            phase_np[c * 24 + 2 * f + 1] = np.pi / 2
    sp = jnp.asarray(np.stack([scale_np, phase_np], axis=1))
    ab = jnp.stack([p['alpha'].astype(_F32),
                    p['beta'].astype(_F32)]).reshape(1, 2)

    W1 = p['enc_W1']
    W1d, W1c = W1[:, :Cin], W1[:, Cin:]
    wt1 = jnp.zeros((Cin, _CH), _F32).at[:, :24].set(W1d.T)
    wt1 = wt1.at[:, 24:24 + Cin].set(jnp.eye(Cin, dtype=_F32))
    wv1 = jnp.zeros((Cin, _CH), _F32).at[:, :24].set((W1c - W1d).T)
    bv1 = jnp.zeros((_CH,), _F32).at[:24].set(p['enc_b1'])

    table1, v1rows = _tc_ab(posT12, _blockdiag4(wt1), _blockdiag4(wv1),
                            jnp.tile(bv1, 4).reshape(1, 128))
    a1rows, aa, knn_flat = _sc_edge_gather(
        table1.reshape(B * N, _CH), idx_rows,
        v1rows.reshape(B * N // 4, 128), B, N, K, emit_extras=True)

    W2 = p['enc_W2']
    W2d, W2c = W2[:, :24], W2[:, 24:]
    wt2 = jnp.zeros((_CH, _CH), _F32).at[:24, :24].set(W2d.T)
    wv2 = jnp.zeros((_CH, _CH), _F32).at[:24, :24].set((W2c - W2d).T)
    bv2 = jnp.zeros((_CH,), _F32).at[:24].set(p['enc_b2'])

    table2, v2rows = _tc_ab(a1rows.reshape(B, N // 4, 128),
                            _blockdiag4(wt2), _blockdiag4(wv2),
                            jnp.tile(bv2, 4).reshape(1, 128))
    gg = _sc_edge_gather(
        table2.reshape(B * N, _CH), idx_rows,
        v2rows.reshape(B * N // 4, 128), B, N, K, emit_extras=False)

    weights = [
        p['mlp1_W1'], p['mlp1_b1'].reshape(-1, 1),
        p['mlp1_W2'], p['mlp1_b2'].reshape(-1, 1),
        p['lin0_W'], p['lin0_b'].reshape(-1, 1),
        p['lin1_W'], p['lin1_b'].reshape(-1, 1),
        p['lin2_W'], p['lin2_b'].reshape(-1, 1),
        p['lin3_W'], p['lin3_b'].reshape(-1, 1),
        p['lin4_W'], p['lin4_b'].reshape(-1, 1),
        p['conv1_W'], p['conv1_b'].reshape(-1, 1),
        p['p1_W'], p['p1_b'].reshape(-1, 1),
        p['p2_W'], p['p2_b'].reshape(-1, 1),
        p['p3_W'], p['p3_b'].reshape(-1, 1),
        p['p4_W'], p['p4_b'].reshape(-1, 1),
        p['p5_W'], p['p5_b'].reshape(-1, 1),
        p['conv2_W'], p['conv2_b'].reshape(-1, 1),
        p['conv3_W'], p['conv3_b'].reshape(-1, 1),
    ]
    out, dist_w, y_out, x_embed = _tc_c1(pos, gg, sp, ab, weights)
    knn_pos_out = knn_flat.reshape(B, Cin, N, K)
    return (out, dist_w, y_out, aa, gg, knn_pos_out, x_embed)
